# baseline (speedup 1.0000x reference)
import functools

import jax
import jax.numpy as jnp
from jax import lax
from jax.experimental import pallas as pl
from jax.experimental.pallas import tpu as pltpu
from jax.experimental.pallas import tpu_sc as plsc

N = 16384
D = 2048

_NC = 2
_NS = 16
_NW = _NC * _NS
_RW = N // _NW
_CH = 16
_NCHUNK = _RW // _CH


def _sc_body(t_hbm, nl_hbm, m2l_hbm, xp_hbm, x_hbm, out_hbm,
             t_v, nl_v, m2l_v, xp_v, x_v, out_v, mat_v):
    wid = lax.axis_index("s") * _NC + lax.axis_index("c")
    base = wid * _RW
    pltpu.sync_copy(t_hbm.at[pl.ds(base, _RW)], t_v)
    pltpu.sync_copy(nl_hbm, nl_v)
    pltpu.sync_copy(m2l_hbm, m2l_v)
    neg_logs = nl_v[...]
    m2logs = m2l_v[...]
    lane = lax.iota(jnp.int32, 16)

    def chunk_body(g, carry):
        pltpu.sync_copy(xp_hbm.at[pl.ds(base + g * _CH, _CH)], xp_v)
        pltpu.sync_copy(x_hbm.at[pl.ds(base + g * _CH, _CH)], x_v)
        res = jnp.zeros((16,), jnp.float32)
        for r in range(_CH):
            def kstep(k, acc):
                for u in range(4):
                    off = k * 64 + u * 16
                    dv = xp_v[r, pl.ds(off, 16)] - x_v[r, pl.ds(off, 16)]
                    acc = acc + dv * dv
                return acc
            acc = lax.fori_loop(0, D // 64, kstep, jnp.zeros((16,), jnp.float32))
            lane_c = lax.iota(jnp.int32, 16)
            dnums = lax.GatherDimensionNumbers(
                offset_dims=(), collapsed_slice_dims=(0,), start_index_map=(0,))
            for kk in (8, 4, 2, 1):
                idx = jnp.bitwise_xor(lane_c, kk)
                shuf = lax.gather(
                    acc, idx[:, None], dnums, (1,),
                    indices_are_sorted=False, unique_indices=False,
                    mode=lax.GatherScatterMode.PROMISE_IN_BOUNDS)
                acc = acc + shuf
            res = jnp.where(lane_c == r, acc, res)
        t16 = t_v[pl.ds(g * _CH, _CH)]
        scale = neg_logs * jnp.exp(m2logs * t16)
        out_v[pl.ds(g * _CH, _CH)] = scale * res
        return carry

    lax.fori_loop(0, _NCHUNK, chunk_body, jnp.int32(0))
    pltpu.sync_copy(out_v, out_hbm.at[pl.ds(base, _RW)])


def _sc_loss(t_flat, neg_logs, m2logs, x_pred, x):
    mesh = plsc.VectorSubcoreMesh(core_axis_name="c", subcore_axis_name="s")
    run = functools.partial(
        pl.kernel,
        mesh=mesh,
        out_type=jax.ShapeDtypeStruct((N,), jnp.float32),
        scratch_types=[
            pltpu.VMEM((_RW,), jnp.float32),
            pltpu.VMEM((16,), jnp.float32),
            pltpu.VMEM((16,), jnp.float32),
            pltpu.VMEM((_CH, D), jnp.float32),
            pltpu.VMEM((_CH, D), jnp.float32),
            pltpu.VMEM((_RW,), jnp.float32),
            pltpu.VMEM((_CH * 16,), jnp.float32),
        ],
    )(_sc_body)
    return run(t_flat, neg_logs, m2logs, x_pred, x)


def kernel(t, sigma1, x_pred, x):
    logs = jnp.log(sigma1[0])
    neg_logs = jnp.full((16,), -logs, jnp.float32)
    m2logs = jnp.full((16,), -2.0 * logs, jnp.float32)
    return _sc_loss(t.reshape(-1), neg_logs, m2logs, x_pred, x)


# speedup vs baseline: 1.6786x; 1.6786x over previous
import functools

import jax
import jax.numpy as jnp
from jax import lax
from jax.experimental import pallas as pl
from jax.experimental.pallas import tpu as pltpu
from jax.experimental.pallas import tpu_sc as plsc

N = 16384
D = 2048

_NC = 2
_NS = 16
_NW = _NC * _NS
_RW = N // _NW
_CH = 8
_NCHUNK = _RW // _CH


def _row_sum_to_lanes(acc, r):
    lane_c = lax.iota(jnp.int32, 16)
    dnums = lax.GatherDimensionNumbers(
        offset_dims=(), collapsed_slice_dims=(0,), start_index_map=(0,))
    for kk in (8, 4, 2, 1):
        idx = jnp.bitwise_xor(lane_c, kk)
        shuf = lax.gather(
            acc, idx[:, None], dnums, (1,),
            indices_are_sorted=False, unique_indices=False,
            mode=lax.GatherScatterMode.PROMISE_IN_BOUNDS)
        acc = acc + shuf
    return lane_c == r, acc


def _sc_body(t_hbm, nl_hbm, m2l_hbm, xp_hbm, x_hbm, out_hbm,
             t_v, nl_v, m2l_v, xp_v, x_v, out_v, sem0, sem1):
    wid = lax.axis_index("s") * _NC + lax.axis_index("c")
    base = wid * _RW
    pltpu.sync_copy(t_hbm.at[pl.ds(base, _RW)], t_v)
    pltpu.sync_copy(nl_hbm, nl_v)
    pltpu.sync_copy(m2l_hbm, m2l_v)
    neg_logs = nl_v[...]
    m2logs = m2l_v[...]
    sems = (sem0, sem1)
    last = _NCHUNK - 1

    def start(c, b):
        row = base + jnp.minimum(c, last) * _CH
        pltpu.async_copy(xp_hbm.at[pl.ds(row, _CH)], xp_v.at[b], sems[b])
        pltpu.async_copy(x_hbm.at[pl.ds(row, _CH)], x_v.at[b], sems[b])

    def drain(b):
        pltpu.make_async_copy(
            xp_hbm.at[pl.ds(base, _CH)], xp_v.at[b], sems[b]).wait()
        pltpu.make_async_copy(
            x_hbm.at[pl.ds(base, _CH)], x_v.at[b], sems[b]).wait()

    start(0, 0)
    start(1, 1)

    def super_body(g, carry):
        res = jnp.zeros((16,), jnp.float32)
        for b in range(2):
            c = 2 * g + b
            drain(b)
            for r in range(_CH):
                def kstep(k, acc):
                    for u in range(8):
                        off = k * 128 + u * 16
                        dv = (xp_v[b, r, pl.ds(off, 16)]
                              - x_v[b, r, pl.ds(off, 16)])
                        acc = acc + dv * dv
                    return acc
                acc = lax.fori_loop(0, D // 128, kstep,
                                    jnp.zeros((16,), jnp.float32))
                is_r, tot = _row_sum_to_lanes(acc, r + _CH * b)
                res = jnp.where(is_r, tot, res)
            start(c + 2, b)
        t16 = t_v[pl.ds(g * 16, 16)]
        scale = neg_logs * jnp.exp(m2logs * t16)
        out_v[pl.ds(g * 16, 16)] = scale * res
        return carry

    lax.fori_loop(0, _NCHUNK // 2, super_body, jnp.int32(0))
    drain(0)
    drain(1)
    pltpu.sync_copy(out_v, out_hbm.at[pl.ds(base, _RW)])


def _sc_loss(t_flat, neg_logs, m2logs, x_pred, x):
    mesh = plsc.VectorSubcoreMesh(core_axis_name="c", subcore_axis_name="s")
    run = functools.partial(
        pl.kernel,
        mesh=mesh,
        out_type=jax.ShapeDtypeStruct((N,), jnp.float32),
        scratch_types=[
            pltpu.VMEM((_RW,), jnp.float32),
            pltpu.VMEM((16,), jnp.float32),
            pltpu.VMEM((16,), jnp.float32),
            pltpu.VMEM((2, _CH, D), jnp.float32),
            pltpu.VMEM((2, _CH, D), jnp.float32),
            pltpu.VMEM((_RW,), jnp.float32),
            pltpu.SemaphoreType.DMA,
            pltpu.SemaphoreType.DMA,
        ],
    )(_sc_body)
    return run(t_flat, neg_logs, m2logs, x_pred, x)


def kernel(t, sigma1, x_pred, x):
    logs = jnp.log(sigma1[0])
    neg_logs = jnp.full((16,), -logs, jnp.float32)
    m2logs = jnp.full((16,), -2.0 * logs, jnp.float32)
    return _sc_loss(t.reshape(-1), neg_logs, m2logs, x_pred, x)
